# baseline (device time: 138387 ns/iter reference)
import jax
import jax.numpy as jnp
from jax import lax
from jax.experimental import pallas as pl
from jax.experimental.pallas import tpu as pltpu

N_DEV = 4
LANE = 128


def _cmpex(v, idx, j, up):
    vm = jnp.roll(v, -j, axis=0)
    vp = jnp.roll(v, j, axis=0)
    lower = (idx & j) == 0
    partner = jnp.where(lower, vm, vp)
    mn = jnp.minimum(v, partner)
    mx = jnp.maximum(v, partner)
    take_min = lower == up
    return jnp.where(take_min, mn, mx)


def _bitonic_sort(v, idx, desc, jmin=1, jmax=1 << 30):
    m = v.shape[0]
    k = 2
    while k <= m:
        j = k // 2
        while j >= 1:
            if jmin <= j <= jmax:
                up = jnp.logical_xor((idx & k) == 0, desc)
                v = _cmpex(v, idx, j, up)
            j //= 2
        k *= 2
    return v


def _local_stages(v, idx, up, j_start):
    j = j_start
    while j >= 1:
        v = _cmpex(v, idx, j, up)
        j //= 2
    return v


def _xmerge(take_min, a, b):
    return jnp.where(take_min, jnp.minimum(a, b), jnp.maximum(a, b))


def kernel(x, *, only_presort=False, jmin=1, jmax=1 << 30):
    m_per, n = x.shape
    nc = n // LANE
    bf16 = jnp.bfloat16

    def body(x_ref, out_ref, s_ref, r1_ref, a_ref, ra_ref,
             send_sems, recv_sems):
        my = lax.axis_index("i")
        peers = [my ^ d for d in (1, 2, 3)]
        q1 = peers[0]

        barrier_sem = pltpu.get_barrier_semaphore()
        for nbr in peers:
            pl.semaphore_signal(
                barrier_sem, inc=1,
                device_id=(nbr,), device_id_type=pl.DeviceIdType.MESH,
            )
        pl.semaphore_wait(barrier_sem, 3)

        desc = (my & 1) == 1
        up_a = (my & 2) == 0
        tm_a = ((my & 1) == 0) == up_a
        tm_b = (my & 2) == 0
        tm_c = (my & 1) == 0
        idx_loc = lax.broadcasted_iota(jnp.int32, (m_per, 1), 0)

        def xchg(src_slice, dst_slice, r, c, peer):
            return pltpu.make_async_remote_copy(
                src_ref=src_slice,
                dst_ref=dst_slice,
                send_sem=send_sems.at[r, c],
                recv_sem=recv_sems.at[r, c],
                device_id=(peer,),
                device_id_type=pl.DeviceIdType.MESH,
            )

        def x1(c):
            return xchg(s_ref.at[c], r1_ref.at[c], 0, c, q1)

        def bcast(c, d):
            return xchg(a_ref.at[c], ra_ref.at[c, d - 1], d, c, peers[d - 1])

        def p0(c, carry):
            v = x_ref[:, pl.ds(c * LANE, LANE)].astype(bf16)
            s_ref[c] = _bitonic_sort(v, idx_loc, desc, jmin, jmax)
            if not only_presort:
                x1(c).start()
            return carry

        lax.fori_loop(0, nc, p0, 0)

        if only_presort:
            def pout(c, carry):
                out_ref[:, pl.ds(c * LANE, LANE)] = s_ref[c]
                return carry

            lax.fori_loop(0, nc, pout, 0)
            return

        def p1(c, carry):
            x1(c).wait_recv()
            w = _xmerge(tm_a, s_ref[c], r1_ref[c])
            w = _local_stages(w, idx_loc, up_a, m_per // 2)
            a_ref[c] = w
            for d in (1, 2, 3):
                bcast(c, d).start()
            return carry

        lax.fori_loop(0, nc, p1, 0)

        def p2(c, carry):
            for d in (1, 2, 3):
                bcast(c, d).wait_recv()
            a1 = ra_ref[c, 0]
            a2 = ra_ref[c, 1]
            a3 = ra_ref[c, 2]
            b_own = _xmerge(tm_b, a_ref[c], a2)
            b_nbr = _xmerge(tm_b, a1, a3)
            w = _xmerge(tm_c, b_own, b_nbr)
            w = _local_stages(w, idx_loc, True, m_per // 2)
            out_ref[:, pl.ds(c * LANE, LANE)] = w
            return carry

        lax.fori_loop(0, nc, p2, 0)

        def drain(c, carry):
            x1(c).wait_send()
            for d in (1, 2, 3):
                bcast(c, d).wait_send()
            return carry

        lax.fori_loop(0, nc, drain, 0)

    blk = (nc, m_per, LANE)
    out = pl.pallas_call(
        body,
        out_shape=jax.ShapeDtypeStruct((m_per, n), bf16),
        in_specs=[pl.BlockSpec(memory_space=pltpu.VMEM)],
        out_specs=pl.BlockSpec(memory_space=pltpu.VMEM),
        scratch_shapes=[
            pltpu.VMEM(blk, bf16),
            pltpu.VMEM(blk, bf16),
            pltpu.VMEM(blk, bf16),
            pltpu.VMEM((nc, 3, m_per, LANE), bf16),
            pltpu.SemaphoreType.DMA((4, nc)),
            pltpu.SemaphoreType.DMA((4, nc)),
        ],
        compiler_params=pltpu.CompilerParams(
            collective_id=0,
            vmem_limit_bytes=60 * 1024 * 1024,
        ),
    )(x)

    return out


# device time: 134251 ns/iter; 1.0308x vs baseline; 1.0308x over previous
import jax
import jax.numpy as jnp
from jax import lax
from jax.experimental import pallas as pl
from jax.experimental.pallas import tpu as pltpu

N_DEV = 4
LANE = 256


def _cmpex(v, idx, j, up):
    vm = jnp.roll(v, -j, axis=0)
    vp = jnp.roll(v, j, axis=0)
    lower = (idx & j) == 0
    partner = jnp.where(lower, vm, vp)
    mn = jnp.minimum(v, partner)
    mx = jnp.maximum(v, partner)
    take_min = lower == up
    return jnp.where(take_min, mn, mx)


def _bitonic_sort(v, idx, desc, jmin=1, jmax=1 << 30):
    m = v.shape[0]
    k = 2
    while k <= m:
        j = k // 2
        while j >= 1:
            if jmin <= j <= jmax:
                up = jnp.logical_xor((idx & k) == 0, desc)
                v = _cmpex(v, idx, j, up)
            j //= 2
        k *= 2
    return v


def _local_stages(v, idx, up, j_start):
    j = j_start
    while j >= 1:
        v = _cmpex(v, idx, j, up)
        j //= 2
    return v


def _xmerge(take_min, a, b):
    return jnp.where(take_min, jnp.minimum(a, b), jnp.maximum(a, b))


def kernel(x, *, only_presort=False, jmin=1, jmax=1 << 30):
    m_per, n = x.shape
    nc = n // LANE
    bf16 = jnp.bfloat16

    def body(x_ref, out_ref, s_ref, r1_ref, a_ref, ra_ref,
             send_sems, recv_sems):
        my = lax.axis_index("i")
        peers = [my ^ d for d in (1, 2, 3)]
        q1 = peers[0]

        barrier_sem = pltpu.get_barrier_semaphore()
        for nbr in peers:
            pl.semaphore_signal(
                barrier_sem, inc=1,
                device_id=(nbr,), device_id_type=pl.DeviceIdType.MESH,
            )
        pl.semaphore_wait(barrier_sem, 3)

        desc = (my & 1) == 1
        up_a = (my & 2) == 0
        tm_a = ((my & 1) == 0) == up_a
        tm_b = (my & 2) == 0
        tm_c = (my & 1) == 0
        idx_loc = lax.broadcasted_iota(jnp.int32, (m_per, 1), 0)

        def xchg(src_slice, dst_slice, r, c, peer):
            return pltpu.make_async_remote_copy(
                src_ref=src_slice,
                dst_ref=dst_slice,
                send_sem=send_sems.at[r, c],
                recv_sem=recv_sems.at[r, c],
                device_id=(peer,),
                device_id_type=pl.DeviceIdType.MESH,
            )

        def x1(c):
            return xchg(s_ref.at[c], r1_ref.at[c], 0, c, q1)

        def bcast(c, d):
            return xchg(a_ref.at[c], ra_ref.at[c, d - 1], d, c, peers[d - 1])

        def p0(c, carry):
            v = x_ref[:, pl.ds(c * LANE, LANE)].astype(bf16)
            s_ref[c] = _bitonic_sort(v, idx_loc, desc, jmin, jmax)
            if not only_presort:
                x1(c).start()
            return carry

        lax.fori_loop(0, nc, p0, 0)

        if only_presort:
            def pout(c, carry):
                out_ref[:, pl.ds(c * LANE, LANE)] = s_ref[c]
                return carry

            lax.fori_loop(0, nc, pout, 0)
            return

        def p1(c, carry):
            x1(c).wait_recv()
            w = _xmerge(tm_a, s_ref[c], r1_ref[c])
            w = _local_stages(w, idx_loc, up_a, m_per // 2)
            a_ref[c] = w
            for d in (1, 2, 3):
                bcast(c, d).start()
            return carry

        lax.fori_loop(0, nc, p1, 0)

        def p2(c, carry):
            for d in (1, 2, 3):
                bcast(c, d).wait_recv()
            a1 = ra_ref[c, 0]
            a2 = ra_ref[c, 1]
            a3 = ra_ref[c, 2]
            b_own = _xmerge(tm_b, a_ref[c], a2)
            b_nbr = _xmerge(tm_b, a1, a3)
            w = _xmerge(tm_c, b_own, b_nbr)
            w = _local_stages(w, idx_loc, True, m_per // 2)
            out_ref[:, pl.ds(c * LANE, LANE)] = w
            return carry

        lax.fori_loop(0, nc, p2, 0)

        def drain(c, carry):
            x1(c).wait_send()
            for d in (1, 2, 3):
                bcast(c, d).wait_send()
            return carry

        lax.fori_loop(0, nc, drain, 0)

    blk = (nc, m_per, LANE)
    out = pl.pallas_call(
        body,
        out_shape=jax.ShapeDtypeStruct((m_per, n), bf16),
        in_specs=[pl.BlockSpec(memory_space=pltpu.VMEM)],
        out_specs=pl.BlockSpec(memory_space=pltpu.VMEM),
        scratch_shapes=[
            pltpu.VMEM(blk, bf16),
            pltpu.VMEM(blk, bf16),
            pltpu.VMEM(blk, bf16),
            pltpu.VMEM((nc, 3, m_per, LANE), bf16),
            pltpu.SemaphoreType.DMA((4, nc)),
            pltpu.SemaphoreType.DMA((4, nc)),
        ],
        compiler_params=pltpu.CompilerParams(
            collective_id=0,
            vmem_limit_bytes=60 * 1024 * 1024,
        ),
    )(x)

    return out
